# baseline (speedup 1.0000x reference)
import numpy as np
import jax
import jax.numpy as jnp
from jax import lax
from jax.experimental import pallas as pl
from jax.experimental.pallas import tpu as pltpu
from jax.experimental.pallas import tpu_sc as plsc

L = 16
N = 14
FIN, HID, FOUT = 256, 128, 64
E = 40
EP = 48
RH = 7


def _splat2(ref, i, j):
    ii = jnp.full((L,), i, jnp.int32)
    jj = jnp.full((L,), j, jnp.int32)
    return plsc.load_gather(ref, [ii, jj])


def _sc_body(src_ref, dst_ref, lut_ref, fea_ref, w1_ref, b1_ref, w2_ref,
             b2_ref, out_ref,
             src_v, dst_v, lut_v, deg_v, dinv_v, a_v, fea_v, w1c_v, b1c_v, afea_v,
             x1stage_v, x1sh, x1_v, w2c_v, b2c_v, ax1_v, outstage_v):
    cid = lax.axis_index("c")
    sid = lax.axis_index("s")
    jc = sid % (HID // L)
    rh = sid // (HID // L)
    base = rh * RH
    f32 = jnp.float32
    iota = lax.iota(jnp.int32, L)
    zero = jnp.zeros((L,), f32)

    pltpu.sync_copy(src_ref, src_v)
    pltpu.sync_copy(dst_ref, dst_v)
    pltpu.sync_copy(lut_ref, lut_v)
    pltpu.sync_copy(fea_ref, fea_v)
    pltpu.sync_copy(w1_ref.at[pl.ds(jc * (FIN * L), FIN * L)], w1c_v)
    pltpu.sync_copy(b1_ref.at[pl.ds(jc * L, L)], b1c_v)

    deg_v[...] = jnp.ones((L,), f32)
    for t in range(EP // L):
        d_idx = dst_v[pl.ds(t * L, L)]
        plsc.addupdate_scatter(deg_v, [d_idx], jnp.ones((L,), f32),
                               mask=(iota + t * L) < E)
    deg_i = jnp.minimum(deg_v[...].astype(jnp.int32), L - 1)
    dinv = plsc.load_gather(lut_v, [deg_i])
    dinv_v[...] = dinv
    for i in range(L):
        a_v[i] = zero
    plsc.addupdate_scatter(a_v, [iota, iota], dinv * dinv, mask=iota < N)
    for t in range(EP // L):
        s_idx = src_v[pl.ds(t * L, L)]
        d_idx = dst_v[pl.ds(t * L, L)]
        nrm = plsc.load_gather(dinv_v, [s_idx]) * plsc.load_gather(dinv_v, [d_idx])
        plsc.addupdate_scatter(a_v, [d_idx, s_idx], nrm,
                               mask=(iota + t * L) < E)

    nc1 = FIN // L
    for io in range(RH):
        i = base + io
        def afea_body(m, accs):
            av = _splat2(a_v, i, m)
            return tuple(accs[c] + av * fea_v[m, pl.ds(c * L, L)]
                         for c in range(nc1))
        accs = lax.fori_loop(0, N, afea_body, (zero,) * nc1)
        for c in range(nc1):
            afea_v[io, pl.ds(c * L, L)] = accs[c]

    b1c = b1c_v[...]
    for io in range(RH):
        def mm1_body(kc, acc):
            v = afea_v[io, pl.ds(kc * L, L)]
            for j in range(L):
                acc = acc + (jnp.full((L,), v[j], f32)
                             * w1c_v[pl.ds((kc * L + j) * L, L)])
            return acc
        acc = lax.fori_loop(0, nc1, mm1_body, zero)
        x1stage_v[pl.ds(io * L, L)] = jnp.maximum(acc + b1c, 0.0)
    for io in range(RH):
        pltpu.sync_copy(x1stage_v.at[pl.ds(io * L, L)],
                        x1sh.at[pl.ds((base + io) * HID + jc * L, L)])

    plsc.subcore_barrier()

    jc2 = cid * 2 + (sid % 2)
    base2 = (sid // 2) * 2

    @pl.when(base2 < N)
    def _phase2():
        pltpu.sync_copy(w2_ref.at[pl.ds(jc2 * (HID * L), HID * L)], w2c_v)
        pltpu.sync_copy(b2_ref.at[pl.ds(jc2 * L, L)], b2c_v)
        pltpu.sync_copy(x1sh, x1_v)
        nc2 = HID // L
        for io in range(2):
            i = base2 + io
            def ax1_body(m, accs):
                av = _splat2(a_v, i, m)
                return tuple(accs[c] + av * x1_v[pl.ds(m * HID + c * L, L)]
                             for c in range(nc2))
            accs = lax.fori_loop(0, N, ax1_body, (zero,) * nc2)
            for c in range(nc2):
                ax1_v[io, pl.ds(c * L, L)] = accs[c]

        b2c = b2c_v[...]
        for io in range(2):
            def mm2_body(kc, acc):
                v = ax1_v[io, pl.ds(kc * L, L)]
                for j in range(L):
                    acc = acc + (jnp.full((L,), v[j], f32)
                                 * w2c_v[pl.ds((kc * L + j) * L, L)])
                return acc
            acc = lax.fori_loop(0, nc2, mm2_body, zero)
            outstage_v[pl.ds(io * L, L)] = acc + b2c
            pltpu.sync_copy(outstage_v.at[pl.ds(io * L, L)],
                            out_ref.at[pl.ds((base2 + io) * FOUT + jc2 * L, L)])


_RSQRT_LUT = np.array([1.0] + [float(i) ** -0.5 for i in range(1, L)],
                      dtype=np.float32)


def kernel(fea, edge_index, W1, b1, W2, b2):
    ei = edge_index.astype(jnp.int32)
    src = jnp.pad(ei[0], (0, EP - E))
    dst = jnp.pad(ei[1], (0, EP - E))
    lut = jnp.asarray(_RSQRT_LUT)
    w1f = W1.reshape(FIN, HID // L, L).transpose(1, 0, 2).reshape(-1)
    w2f = W2.reshape(HID, FOUT // L, L).transpose(1, 0, 2).reshape(-1)

    mesh = plsc.VectorSubcoreMesh(core_axis_name="c", subcore_axis_name="s")
    fn = pl.kernel(
        _sc_body,
        out_type=jax.ShapeDtypeStruct((N * FOUT,), jnp.float32),
        mesh=mesh,
        compiler_params=pltpu.CompilerParams(needs_layout_passes=False),
        scratch_types=[
            pltpu.VMEM((EP,), jnp.int32),
            pltpu.VMEM((EP,), jnp.int32),
            pltpu.VMEM((L,), jnp.float32),
            pltpu.VMEM((L,), jnp.float32),
            pltpu.VMEM((L,), jnp.float32),
            pltpu.VMEM((L, L), jnp.float32),
            pltpu.VMEM((N, FIN), jnp.float32),
            pltpu.VMEM((FIN * L,), jnp.float32),
            pltpu.VMEM((L,), jnp.float32),
            pltpu.VMEM((RH, FIN), jnp.float32),
            pltpu.VMEM((RH * L,), jnp.float32),
            pltpu.VMEM_SHARED((N * HID,), jnp.float32),
            pltpu.VMEM((N * HID,), jnp.float32),
            pltpu.VMEM((HID * L,), jnp.float32),
            pltpu.VMEM((L,), jnp.float32),
            pltpu.VMEM((2, HID), jnp.float32),
            pltpu.VMEM((2 * L,), jnp.float32),
        ],
    )
    out = fn(src, dst, lut, fea, w1f, b1, w2f, b2)
    return out.reshape(N, FOUT)


# speedup vs baseline: 1.1867x; 1.1867x over previous
import numpy as np
import jax
import jax.numpy as jnp
from jax import lax
from jax.experimental import pallas as pl
from jax.experimental.pallas import tpu as pltpu
from jax.experimental.pallas import tpu_sc as plsc

L = 16
N = 14
FIN, HID, FOUT = 256, 128, 64
E = 40
EP = 48
RH = 7


def _splat2(ref, i, j):
    ii = jnp.full((L,), i, jnp.int32)
    jj = jnp.full((L,), j, jnp.int32)
    return plsc.load_gather(ref, [ii, jj])


def _sc_body(src_ref, dst_ref, lut_ref, fea_ref, w1_ref, b1_ref, w2_ref,
             b2_ref, out_ref,
             src_v, dst_v, lut_v, deg_v, dinv_v, a_v, fea_v, w1c_v, b1c_v, afea_v,
             x1stage_v, x1sh, x1_v, w2c_v, b2c_v, ax1_v, outstage_v, sem):
    cid = lax.axis_index("c")
    sid = lax.axis_index("s")
    jc = sid % (HID // L)
    rh = sid // (HID // L)
    base = rh * RH
    f32 = jnp.float32
    iota = lax.iota(jnp.int32, L)
    zero = jnp.zeros((L,), f32)

    jc2 = cid * 2 + (sid % 2)
    base2 = (sid // 2) * 2

    copies = [
        pltpu.async_copy(src_ref, src_v, sem),
        pltpu.async_copy(dst_ref, dst_v, sem),
        pltpu.async_copy(lut_ref, lut_v, sem),
        pltpu.async_copy(fea_ref, fea_v, sem),
        pltpu.async_copy(w1_ref.at[pl.ds(jc * (FIN * L), FIN * L)], w1c_v, sem),
        pltpu.async_copy(b1_ref.at[pl.ds(jc * L, L)], b1c_v, sem),
        pltpu.async_copy(w2_ref.at[pl.ds(jc2 * (HID * L), HID * L)], w2c_v, sem),
        pltpu.async_copy(b2_ref.at[pl.ds(jc2 * L, L)], b2c_v, sem),
    ]
    for h in copies:
        h.wait()

    deg_v[...] = jnp.ones((L,), f32)
    for t in range(EP // L):
        d_idx = dst_v[pl.ds(t * L, L)]
        plsc.addupdate_scatter(deg_v, [d_idx], jnp.ones((L,), f32),
                               mask=(iota + t * L) < E)
    deg_i = jnp.minimum(deg_v[...].astype(jnp.int32), L - 1)
    dinv = plsc.load_gather(lut_v, [deg_i])
    dinv_v[...] = dinv
    for i in range(L):
        a_v[i] = zero
    plsc.addupdate_scatter(a_v, [iota, iota], dinv * dinv, mask=iota < N)
    for t in range(EP // L):
        s_idx = src_v[pl.ds(t * L, L)]
        d_idx = dst_v[pl.ds(t * L, L)]
        nrm = plsc.load_gather(dinv_v, [s_idx]) * plsc.load_gather(dinv_v, [d_idx])
        plsc.addupdate_scatter(a_v, [d_idx, s_idx], nrm,
                               mask=(iota + t * L) < E)

    nc1 = FIN // L
    for io in range(RH):
        i = base + io
        def afea_body(m, accs):
            av = _splat2(a_v, i, m)
            return tuple(accs[c] + av * fea_v[m, pl.ds(c * L, L)]
                         for c in range(nc1))
        accs = lax.fori_loop(0, N, afea_body, (zero,) * nc1)
        for c in range(nc1):
            afea_v[io, pl.ds(c * L, L)] = accs[c]

    b1c = b1c_v[...]
    for io in range(RH):
        def mm1_body(kc, accs):
            accs = list(accs)
            v = afea_v[io, pl.ds(kc * L, L)]
            for j in range(L):
                accs[j % 4] = accs[j % 4] + (jnp.full((L,), v[j], f32)
                                             * w1c_v[pl.ds((kc * L + j) * L, L)])
            return tuple(accs)
        a0, a1, a2, a3 = lax.fori_loop(0, nc1, mm1_body, (zero,) * 4)
        acc = (a0 + a1) + (a2 + a3)
        x1stage_v[pl.ds(io * L, L)] = jnp.maximum(acc + b1c, 0.0)
    for io in range(RH):
        pltpu.sync_copy(x1stage_v.at[pl.ds(io * L, L)],
                        x1sh.at[pl.ds((base + io) * HID + jc * L, L)])

    plsc.subcore_barrier()

    @pl.when(base2 < N)
    def _phase2():
        pltpu.sync_copy(x1sh, x1_v)
        nc2 = HID // L
        for io in range(2):
            i = base2 + io
            def ax1_body(m, accs):
                av = _splat2(a_v, i, m)
                return tuple(accs[c] + av * x1_v[pl.ds(m * HID + c * L, L)]
                             for c in range(nc2))
            accs = lax.fori_loop(0, N, ax1_body, (zero,) * nc2)
            for c in range(nc2):
                ax1_v[io, pl.ds(c * L, L)] = accs[c]

        b2c = b2c_v[...]
        for io in range(2):
            def mm2_body(kc, accs):
                accs = list(accs)
                v = ax1_v[io, pl.ds(kc * L, L)]
                for j in range(L):
                    accs[j % 4] = accs[j % 4] + (jnp.full((L,), v[j], f32)
                                                 * w2c_v[pl.ds((kc * L + j) * L, L)])
                return tuple(accs)
            a0, a1, a2, a3 = lax.fori_loop(0, nc2, mm2_body, (zero,) * 4)
            acc = (a0 + a1) + (a2 + a3)
            outstage_v[pl.ds(io * L, L)] = acc + b2c
            pltpu.sync_copy(outstage_v.at[pl.ds(io * L, L)],
                            out_ref.at[pl.ds((base2 + io) * FOUT + jc2 * L, L)])


_RSQRT_LUT = np.array([1.0] + [float(i) ** -0.5 for i in range(1, L)],
                      dtype=np.float32)


def kernel(fea, edge_index, W1, b1, W2, b2):
    ei = edge_index.astype(jnp.int32)
    src = jnp.pad(ei[0], (0, EP - E))
    dst = jnp.pad(ei[1], (0, EP - E))
    lut = jnp.asarray(_RSQRT_LUT)
    w1f = W1.reshape(FIN, HID // L, L).transpose(1, 0, 2).reshape(-1)
    w2f = W2.reshape(HID, FOUT // L, L).transpose(1, 0, 2).reshape(-1)

    mesh = plsc.VectorSubcoreMesh(core_axis_name="c", subcore_axis_name="s")
    fn = pl.kernel(
        _sc_body,
        out_type=jax.ShapeDtypeStruct((N * FOUT,), jnp.float32),
        mesh=mesh,
        compiler_params=pltpu.CompilerParams(needs_layout_passes=False),
        scratch_types=[
            pltpu.VMEM((EP,), jnp.int32),
            pltpu.VMEM((EP,), jnp.int32),
            pltpu.VMEM((L,), jnp.float32),
            pltpu.VMEM((L,), jnp.float32),
            pltpu.VMEM((L,), jnp.float32),
            pltpu.VMEM((L, L), jnp.float32),
            pltpu.VMEM((N, FIN), jnp.float32),
            pltpu.VMEM((FIN * L,), jnp.float32),
            pltpu.VMEM((L,), jnp.float32),
            pltpu.VMEM((RH, FIN), jnp.float32),
            pltpu.VMEM((RH * L,), jnp.float32),
            pltpu.VMEM_SHARED((N * HID,), jnp.float32),
            pltpu.VMEM((N * HID,), jnp.float32),
            pltpu.VMEM((HID * L,), jnp.float32),
            pltpu.VMEM((L,), jnp.float32),
            pltpu.VMEM((2, HID), jnp.float32),
            pltpu.VMEM((2 * L,), jnp.float32),
            pltpu.SemaphoreType.DMA,
        ],
    )
    out = fn(src, dst, lut, fea, w1f, b1, w2f, b2)
    return out.reshape(N, FOUT)
